# baseline (speedup 1.0000x reference)
import jax
import jax.numpy as jnp
from jax.experimental import pallas as pl
from jax.experimental.pallas import tpu as pltpu

T = 64
D = 1024
F = 1408
E = 16
SCALE = 1.0 / (2.0 ** 0.5)
PREC = jax.lax.Precision.DEFAULT


def _dot(a, b):
    return jax.lax.dot_general(a, b, (((1,), (0,)), ((), ())),
                               precision=PREC,
                               preferred_element_type=jnp.float32)


def _moe_body(x_ref, rw_ref, eg_ref, eu_ref, ed_ref, sg_ref, su_ref, sd_ref,
              out_ref, comb_ref):
    e = pl.program_id(0)

    @pl.when(e == 0)
    def _init():
        x = x_ref[...]
        lg = jax.lax.dot_general(x, rw_ref[...], (((1,), (1,)), ((), ())),
                                 precision=PREC,
                                 preferred_element_type=jnp.float32)
        logits = lg[:, :E]
        iota = jax.lax.broadcasted_iota(jnp.int32, (T, E), 1)
        v1 = jnp.max(logits, axis=-1, keepdims=True)
        i1 = jnp.min(jnp.where(logits == v1, iota, E), axis=-1, keepdims=True)
        masked = jnp.where(iota == i1, -jnp.inf, logits)
        v2 = jnp.max(masked, axis=-1, keepdims=True)
        i2 = jnp.min(jnp.where(masked == v2, iota, E), axis=-1, keepdims=True)
        w1 = jax.nn.sigmoid(v1 - v2)
        w2 = jax.nn.sigmoid(v2 - v1)
        sscore = jax.nn.sigmoid(lg[:, E:E + 1])
        lane = jax.lax.broadcasted_iota(jnp.int32, (T, 128), 1)
        comb_full = (jnp.where(lane == i1, w1, 0.0)
                     + jnp.where(lane == i2, w2, 0.0)
                     + jnp.where(lane == E, sscore, 0.0)) * SCALE
        comb_ref[...] = comb_full
        out_ref[...] = jnp.zeros_like(out_ref)

    xx = x_ref[...]
    lane = jax.lax.broadcasted_iota(jnp.int32, (T, 128), 1)
    comb = comb_ref[...]

    w_e = jnp.sum(jnp.where(lane == e, comb, 0.0), axis=1, keepdims=True)
    h = jax.nn.silu(_dot(xx, eg_ref[0])) * _dot(xx, eu_ref[0]) * w_e
    out_ref[...] += _dot(h, ed_ref[0])

    @pl.when(e == 0)
    def _shared():
        w_s = jnp.sum(jnp.where(lane == E, comb, 0.0), axis=1, keepdims=True)
        hs = jax.nn.silu(_dot(xx, sg_ref[0])) * _dot(xx, su_ref[0]) * w_s
        out_ref[...] += _dot(hs, sd_ref[0])


def kernel(hidden_states, gate_w, shared_expert_gate_w, expert_gate_w,
           expert_up_w, expert_down_w, shared_gate_proj_w, shared_up_w,
           shared_down_w):
    x = hidden_states.reshape(T, D)
    rw = jnp.concatenate(
        [gate_w, shared_expert_gate_w,
         jnp.zeros((7, D), dtype=gate_w.dtype)], axis=0)

    out = pl.pallas_call(
        _moe_body,
        grid=(E,),
        in_specs=[
            pl.BlockSpec((T, D), lambda e: (0, 0)),
            pl.BlockSpec((24, D), lambda e: (0, 0)),
            pl.BlockSpec((1, D, F), lambda e: (e, 0, 0)),
            pl.BlockSpec((1, D, F), lambda e: (e, 0, 0)),
            pl.BlockSpec((1, F, D), lambda e: (e, 0, 0)),
            pl.BlockSpec((1, D, F), lambda e: (0, 0, 0)),
            pl.BlockSpec((1, D, F), lambda e: (0, 0, 0)),
            pl.BlockSpec((1, F, D), lambda e: (0, 0, 0)),
        ],
        out_specs=pl.BlockSpec((T, D), lambda e: (0, 0)),
        out_shape=jax.ShapeDtypeStruct((T, D), jnp.float32),
        scratch_shapes=[pltpu.VMEM((T, 128), jnp.float32)],
        compiler_params=pltpu.CompilerParams(
            dimension_semantics=("arbitrary",),
        ),
    )(x, rw, expert_gate_w, expert_up_w, expert_down_w,
      shared_gate_proj_w, shared_up_w, shared_down_w)
    return out
